# Initial kernel scaffold; baseline (speedup 1.0000x reference)
import jax
import jax.numpy as jnp
from jax.experimental import pallas as pl


def kernel(x, batch, v, lin):
    raise NotImplementedError("write your pallas kernel here")


# speedup vs baseline: 39.1829x; 39.1829x over previous
import jax
import jax.numpy as jnp
from jax.experimental import pallas as pl

_N = 50000
_F = 3
_T = 32
_S = 32
_NSEG = 128
_SCALE = 500.0

_NB = 1024


def _ect_block_kernel(x_ref, b_ref, v_ref, linrep_ref, out_ref):
    i = pl.program_id(0)
    x_blk = x_ref[0]
    v = v_ref[...]
    nh = jax.lax.dot_general(
        x_blk, v, (((1,), (0,)), ((), ())), preferred_element_type=jnp.float32
    )
    nh_tile = jnp.tile(nh, (1, _S))
    z = linrep_ref[...] - nh_tile
    ecc = jax.nn.sigmoid(_SCALE * z)
    seg = b_ref[0, 0]
    onehot = (seg[:, None] == jax.lax.broadcasted_iota(jnp.int32, (_NB, _NSEG), 1))
    contrib = jax.lax.dot_general(
        onehot.astype(jnp.bfloat16),
        ecc.astype(jnp.bfloat16),
        (((0,), (0,)), ((), ())),
        preferred_element_type=jnp.float32,
    )

    @pl.when(i == 0)
    def _():
        out_ref[...] = jnp.zeros_like(out_ref)

    out_ref[...] += contrib


def kernel(x, batch, v, lin):
    n_pad = ((_N + _NB - 1) // _NB) * _NB
    g = n_pad // _NB
    xp = jnp.pad(x, ((0, n_pad - _N), (0, 0))).reshape(g, _NB, _F)
    bp = jnp.pad(batch, (0, n_pad - _N), constant_values=_NSEG)
    bp = bp.reshape(g, 1, _NB)
    linrep = jnp.repeat(lin, _T).reshape(1, _S * _T)

    out = pl.pallas_call(
        _ect_block_kernel,
        grid=(g,),
        in_specs=[
            pl.BlockSpec((1, _NB, _F), lambda i: (i, 0, 0)),
            pl.BlockSpec((1, 1, _NB), lambda i: (i, 0, 0)),
            pl.BlockSpec((_F, _T), lambda i: (0, 0)),
            pl.BlockSpec((1, _S * _T), lambda i: (0, 0)),
        ],
        out_specs=pl.BlockSpec((_NSEG, _S * _T), lambda i: (0, 0)),
        out_shape=jax.ShapeDtypeStruct((_NSEG, _S * _T), jnp.float32),
    )(xp, bp, v, linrep)
    return out.reshape(_NSEG, _S, _T)
